# baseline (speedup 1.0000x reference)
import jax
import jax.numpy as jnp
from jax import lax
from jax.experimental import pallas as pl
from jax.experimental.pallas import tpu as pltpu
from jax.experimental.pallas import tpu_sc as plsc

N = 10000
E = 320000
IN = 128
HID = 16
HEADS = 8
OUT = 40

NC = 2
NS = 16
NW = NC * NS
C = 80
NCHUNKS = E // C
NK = NCHUNKS // NW
RPT = 624
TAIL0 = RPT * NS
TAILN = N - TAIL0
ACC2W = 48
RB = 2000

_SC_PARAMS = pltpu.CompilerParams(needs_layout_passes=False,
                                  use_tc_tiling_on_sc=False)


def _shuffle16(v, idx16):
    dnums = lax.GatherDimensionNumbers(
        offset_dims=(), collapsed_slice_dims=(0,), start_index_map=(0,))
    return lax.gather(v, idx16[:, None], dnums, (1,),
                      mode=lax.GatherScatterMode.PROMISE_IN_BOUNDS)


def _zero_acc(sid, buf, acc_sh):
    zero16 = jnp.zeros((16,), jnp.float32)
    width = buf.shape[1]

    def zbuf(i, _):
        for k in range(width // 16):
            buf[i, pl.ds(16 * k, 16)] = zero16
        return 0
    lax.fori_loop(0, C, zbuf, 0)

    row0 = sid * RPT
    for off, ln in ((0, 80), (80, 80), (160, 80), (240, 80),
                    (320, 80), (400, 80), (480, 80), (560, 64)):
        pltpu.sync_copy(buf.at[pl.ds(0, ln)], acc_sh.at[pl.ds(row0 + off, ln)])

    @pl.when(sid == NS - 1)
    def _():
        pltpu.sync_copy(buf.at[pl.ds(0, TAILN)], acc_sh.at[pl.ds(TAIL0, TAILN)])


def _drain_acc(sid, acc_sh, out_slab):
    row0 = sid * RPT
    pltpu.sync_copy(acc_sh.at[pl.ds(row0, RPT)], out_slab.at[pl.ds(row0, RPT)])

    @pl.when(sid == NS - 1)
    def _():
        pltpu.sync_copy(acc_sh.at[pl.ds(TAIL0, TAILN)],
                        out_slab.at[pl.ds(TAIL0, TAILN)])


def _tc_a(x_ref, w1_ref, sd_ref, h_ref, t_ref, amax_ref):
    h = jnp.dot(x_ref[...], w1_ref[...], preferred_element_type=jnp.float32)
    h_ref[...] = h
    t = jnp.dot(h, sd_ref[...], preferred_element_type=jnp.float32)
    t_ref[...] = t

    @pl.when(pl.program_id(0) == 0)
    def _():
        amax_ref[...] = jnp.full((1, 16), -1e30, jnp.float32)

    amax_ref[...] = jnp.maximum(amax_ref[...], jnp.max(t, axis=0, keepdims=True))


def _run_a(x, W1, SD):
    return pl.pallas_call(
        _tc_a,
        grid=(N // RB,),
        in_specs=[
            pl.BlockSpec((RB, IN), lambda i: (i, 0)),
            pl.BlockSpec((IN, IN), lambda i: (0, 0)),
            pl.BlockSpec((IN, 16), lambda i: (0, 0)),
        ],
        out_specs=[
            pl.BlockSpec((RB, IN), lambda i: (i, 0)),
            pl.BlockSpec((RB, 16), lambda i: (i, 0)),
            pl.BlockSpec((1, 16), lambda i: (0, 0)),
        ],
        out_shape=[
            jax.ShapeDtypeStruct((N, IN), jnp.float32),
            jax.ShapeDtypeStruct((N, 16), jnp.float32),
            jax.ShapeDtypeStruct((1, 16), jnp.float32),
        ],
    )(x, W1, SD)


def _sc_edge1(src_r, dst_r, t1_r, h1_r, adup_r, accf_r, acce_r,
              ids_s3, ids_d3, isem,
              gs_a, gd_a, hrow_a, exh_a, exb_a, sem_a, ssem_a,
              gs_b, gd_b, hrow_b, exh_b, exb_b, sem_b, ssem_b,
              adup_v, accf_sh, acce_sh):
    cid = lax.axis_index("c")
    sid = lax.axis_index("s")
    wid = sid * NC + cid
    g0 = wid * NK

    _zero_acc(sid, hrow_a, accf_sh)
    _zero_acc(sid, exh_a, acce_sh)

    iota = lax.iota(jnp.int32, 16)
    lo8 = iota < 8
    col_s = iota & 7
    col_d = col_s + 8
    hi8 = (iota >= 8).astype(jnp.int32)
    pltpu.sync_copy(adup_r, adup_v)
    adv = _shuffle16(adup_v[0], col_s)
    plsc.subcore_barrier()

    bufs_a = (gs_a, gd_a, hrow_a, exh_a, exb_a, sem_a, ssem_a)
    bufs_b = (gs_b, gd_b, hrow_b, exh_b, exb_b, sem_b, ssem_b)

    def fire_ids(g):
        slot = lax.rem(g, 3)
        pltpu.async_copy(src_r.at[pl.ds(g0 + g, 1)],
                         ids_s3.at[pl.ds(slot, 1)], isem)
        pltpu.async_copy(dst_r.at[pl.ds(g0 + g, 1)],
                         ids_d3.at[pl.ds(slot, 1)], isem)

    def wait_ids():
        pltpu.make_async_copy(src_r.at[pl.ds(0, 1)],
                              ids_s3.at[pl.ds(0, 1)], isem).wait()
        pltpu.make_async_copy(dst_r.at[pl.ds(0, 1)],
                              ids_d3.at[pl.ds(0, 1)], isem).wait()

    def fire_gathers(g, bufs):
        gs, gd, hrow, _, _, sem, _ = bufs
        slot = lax.rem(g, 3)
        pltpu.async_copy(t1_r.at[ids_s3.at[slot, 0]], gs, sem)
        pltpu.async_copy(t1_r.at[ids_d3.at[slot, 0]], gd, sem)
        pltpu.async_copy(h1_r.at[ids_s3.at[slot, 0]], hrow, sem)

    def wait_gathers(bufs):
        gs, gd, hrow, _, _, sem, _ = bufs
        pltpu.make_async_copy(t1_r.at[ids_s3.at[0, 0]], gs, sem).wait()
        pltpu.make_async_copy(t1_r.at[ids_d3.at[0, 0]], gd, sem).wait()
        pltpu.make_async_copy(h1_r.at[ids_s3.at[0, 0]], hrow, sem).wait()

    def fire_scatter(g, bufs):
        _, _, hrow, exh, _, _, ssem = bufs
        slot = lax.rem(g, 3)
        pltpu.async_copy(hrow, accf_sh.at[ids_d3.at[slot, 0]], ssem, add=True)
        pltpu.async_copy(exh, acce_sh.at[ids_d3.at[slot, 0]], ssem, add=True)

    def wait_scatter(bufs):
        _, _, hrow, exh, _, _, ssem = bufs
        pltpu.make_async_copy(hrow, accf_sh.at[ids_d3.at[0, 0]], ssem).wait()
        pltpu.make_async_copy(exh, acce_sh.at[ids_d3.at[0, 0]], ssem).wait()

    def compute(bufs):
        gs, gd, hrow, exh, exb, _, _ = bufs

        def ex_pair(jp):
            row_idx = 2 * jp + hi8
            s_v = plsc.load_gather(gs, [row_idx, col_s])
            d_v = plsc.load_gather(gd, [row_idx, col_d])
            e = s_v + d_v
            el = jnp.maximum(e, 0.2 * e)
            ad = adv + d_v
            m = jnp.maximum(ad, 0.2 * ad)
            exv = jnp.exp(el - m)
            exb[pl.ds(16 * jp, 16)] = exv
            exh[2 * jp, pl.ds(0, 16)] = jnp.where(lo8, exv, 0.0)
            shuf = _shuffle16(exv, col_d)
            exh[2 * jp + 1, pl.ds(0, 16)] = jnp.where(lo8, shuf, 0.0)

        @plsc.parallel_loop(0, C // 2, unroll=4)
        def _(jp):
            ex_pair(jp)

        def m_pair(jp):
            ex16 = exb[pl.ds(16 * jp, 16)]
            ia = 2 * jp
            ib = 2 * jp + 1
            for k2 in range(8):
                hrow[ia, pl.ds(16 * k2, 16)] = (
                    hrow[ia, pl.ds(16 * k2, 16)] * ex16[k2])
                hrow[ib, pl.ds(16 * k2, 16)] = (
                    hrow[ib, pl.ds(16 * k2, 16)] * ex16[8 + k2])

        @plsc.parallel_loop(0, C // 2, unroll=4)
        def _(jp):
            m_pair(jp)

    pltpu.sync_copy(src_r.at[pl.ds(g0, 1)], ids_s3.at[pl.ds(0, 1)])
    pltpu.sync_copy(dst_r.at[pl.ds(g0, 1)], ids_d3.at[pl.ds(0, 1)])
    fire_gathers(0, bufs_a)
    fire_ids(1)

    def process(g, cur, oth, first):
        wait_gathers(cur)
        if not first:
            wait_scatter(oth)

        @pl.when(g + 1 < NK)
        def _():
            wait_ids()
            fire_gathers(g + 1, oth)

        @pl.when(g + 2 < NK)
        def _():
            fire_ids(g + 2)
        compute(cur)
        fire_scatter(g, cur)

    def step(q, _):
        ge = 2 * q
        go = 2 * q + 1

        @pl.when(q == 0)
        def _():
            process(0, bufs_a, bufs_b, True)

        @pl.when(q > 0)
        def _():
            process(ge, bufs_a, bufs_b, False)

        @pl.when(go < NK)
        def _():
            process(go, bufs_b, bufs_a, False)
        return 0

    lax.fori_loop(0, (NK + 1) // 2, step, 0)
    wait_scatter(bufs_a)

    plsc.subcore_barrier()
    _drain_acc(sid, accf_sh, accf_r.at[cid])
    _drain_acc(sid, acce_sh, acce_r.at[cid])


def _run_sc1(src3, dst3, t1, h1, adup):
    mesh = plsc.VectorSubcoreMesh(core_axis_name="c", subcore_axis_name="s",
                                  num_cores=NC, num_subcores=NS)
    dbuf = [
        pltpu.VMEM((C, 16), jnp.float32),
        pltpu.VMEM((C, 16), jnp.float32),
        pltpu.VMEM((C, IN), jnp.float32),
        pltpu.VMEM((C, 16), jnp.float32),
        pltpu.VMEM((C * 8,), jnp.float32),
        pltpu.SemaphoreType.DMA,
        pltpu.SemaphoreType.DMA,
    ]
    f = pl.kernel(
        _sc_edge1,
        out_type=[
            jax.ShapeDtypeStruct((NC, N, IN), jnp.float32),
            jax.ShapeDtypeStruct((NC, N, 16), jnp.float32),
        ],
        mesh=mesh,
        compiler_params=_SC_PARAMS,
        scratch_types=[
            pltpu.VMEM((3, 1, C), jnp.int32),
            pltpu.VMEM((3, 1, C), jnp.int32),
            pltpu.SemaphoreType.DMA,
            *dbuf, *dbuf,
            pltpu.VMEM((1, 16), jnp.float32),
            pltpu.VMEM_SHARED((N, IN), jnp.float32),
            pltpu.VMEM_SHARED((N, 16), jnp.float32),
        ],
    )
    return f(src3, dst3, t1, h1, adup)


def _tc_c(accf_ref, acce_ref, b1_ref, w2p_ref, a2p_ref, h2_ref, t2_ref, amax_ref):
    a = accf_ref[0] + accf_ref[1]
    den8 = (acce_ref[0] + acce_ref[1])[:, :8]
    r8 = lax.broadcasted_iota(jnp.int32, (8, 128), 0)
    c8 = lax.broadcasted_iota(jnp.int32, (8, 128), 1)
    expand = (r8 == c8 // 16).astype(jnp.float32)
    denx = jnp.dot(den8, expand, preferred_element_type=jnp.float32)
    o = a / (denx + 1e-16) + b1_ref[...]
    o = jnp.where(o > 0, o, jnp.exp(jnp.minimum(o, 0.0)) - 1.0)
    h2 = jnp.dot(o, w2p_ref[...], preferred_element_type=jnp.float32)
    c48 = (lax.broadcasted_iota(jnp.int32, (1, ACC2W), 1) == 40).astype(jnp.float32)
    h2 = h2 + c48
    h2_ref[...] = h2
    t2 = jnp.dot(h2, a2p_ref[...], preferred_element_type=jnp.float32)
    t2_ref[...] = t2

    @pl.when(pl.program_id(0) == 0)
    def _():
        amax_ref[...] = jnp.full((1, 16), -1e30, jnp.float32)

    amax_ref[...] = jnp.maximum(amax_ref[...], jnp.max(t2, axis=0, keepdims=True))


def _run_c(accf, acce, b1, W2p, att2p):
    return pl.pallas_call(
        _tc_c,
        grid=(N // RB,),
        in_specs=[
            pl.BlockSpec((NC, RB, IN), lambda i: (0, i, 0)),
            pl.BlockSpec((NC, RB, 16), lambda i: (0, i, 0)),
            pl.BlockSpec((1, 128), lambda i: (0, 0)),
            pl.BlockSpec((128, ACC2W), lambda i: (0, 0)),
            pl.BlockSpec((ACC2W, 16), lambda i: (0, 0)),
        ],
        out_specs=[
            pl.BlockSpec((RB, ACC2W), lambda i: (i, 0)),
            pl.BlockSpec((RB, 16), lambda i: (i, 0)),
            pl.BlockSpec((1, 16), lambda i: (0, 0)),
        ],
        out_shape=[
            jax.ShapeDtypeStruct((N, ACC2W), jnp.float32),
            jax.ShapeDtypeStruct((N, 16), jnp.float32),
            jax.ShapeDtypeStruct((1, 16), jnp.float32),
        ],
    )(accf, acce, b1, W2p, att2p)


C2 = 128
NCHUNKS2 = E // C2
NK2MIN = NCHUNKS2 // NW
NEXTRA2 = NCHUNKS2 - NK2MIN * NW
NK2MAX = NK2MIN + 1


def _sc_edge2(src_r, dst_r, as2_r, ad2_r, a2dup_r, h2_r, acc_r,
              ids_s3, ids_d3, as2_t, ad2_t,
              h2b_a, exb2_a, sem_a, ssem_a,
              h2b_b, exb2_b, sem_b, ssem_b,
              a2v, acc_sh):
    cid = lax.axis_index("c")
    sid = lax.axis_index("s")
    wid = sid * NC + cid
    g0 = NK2MIN * wid + jnp.minimum(wid, NEXTRA2)
    nk = NK2MIN + jnp.where(wid < NEXTRA2, 1, 0)

    _zero_acc(sid, h2b_a, acc_sh)
    pltpu.sync_copy(src_r.at[pl.ds(g0, NK2MIN)], ids_s3.at[pl.ds(0, NK2MIN)])
    pltpu.sync_copy(dst_r.at[pl.ds(g0, NK2MIN)], ids_d3.at[pl.ds(0, NK2MIN)])

    @pl.when(nk > NK2MIN)
    def _():
        pltpu.sync_copy(src_r.at[pl.ds(g0 + NK2MIN, 1)],
                        ids_s3.at[pl.ds(NK2MIN, 1)])
        pltpu.sync_copy(dst_r.at[pl.ds(g0 + NK2MIN, 1)],
                        ids_d3.at[pl.ds(NK2MIN, 1)])
    pltpu.sync_copy(as2_r, as2_t)
    pltpu.sync_copy(ad2_r, ad2_t)
    pltpu.sync_copy(a2dup_r, a2v)
    plsc.subcore_barrier()

    a2 = _shuffle16(a2v[0], jnp.zeros((16,), jnp.int32))
    bufs_a = (h2b_a, exb2_a, sem_a, ssem_a)
    bufs_b = (h2b_b, exb2_b, sem_b, ssem_b)

    def fire_gathers(g, bufs):
        h2b, _, sem, _ = bufs
        pltpu.async_copy(h2_r.at[ids_s3.at[g, 0]], h2b, sem)

    def wait_gathers(bufs):
        h2b, _, sem, _ = bufs
        pltpu.make_async_copy(h2_r.at[ids_s3.at[0, 0]], h2b, sem).wait()

    def fire_scatter(g, bufs):
        h2b, _, _, ssem = bufs
        pltpu.async_copy(h2b, acc_sh.at[ids_d3.at[g, 0]], ssem, add=True)

    def wait_scatter(bufs):
        h2b, _, _, ssem = bufs
        pltpu.make_async_copy(h2b, acc_sh.at[ids_d3.at[0, 0]], ssem).wait()

    def compute_scale(g, bufs):
        h2b, exb2, _, _ = bufs

        def exloop(j):
            sv = ids_s3[g, 0, pl.ds(16 * j, 16)]
            dv = ids_d3[g, 0, pl.ds(16 * j, 16)]
            s = plsc.load_gather(as2_t, [sv])
            d = plsc.load_gather(ad2_t, [dv])
            e = s + d
            el = jnp.maximum(e, 0.2 * e)
            ad = a2 + d
            m = jnp.maximum(ad, 0.2 * ad)
            exb2[pl.ds(16 * j, 16)] = jnp.exp(el - m)
        plsc.parallel_loop(0, C2 // 16, unroll=4)(exloop)

        @plsc.parallel_loop(0, C2 // 16, unroll=4)
        def _(j):
            ex16 = exb2[pl.ds(16 * j, 16)]
            for k2 in range(16):
                i = 16 * j + k2
                sc = ex16[k2]
                for b in range(ACC2W // 16):
                    h2b[i, pl.ds(16 * b, 16)] = h2b[i, pl.ds(16 * b, 16)] * sc

    fire_gathers(0, bufs_a)

    def process(g, cur, oth, first):
        wait_gathers(cur)
        if not first:
            wait_scatter(oth)

        @pl.when(g + 1 < nk)
        def _():
            fire_gathers(g + 1, oth)
        compute_scale(g, cur)
        fire_scatter(g, cur)

    def step(q, _):
        ge = 2 * q
        go = 2 * q + 1

        @pl.when(q == 0)
        def _():
            process(0, bufs_a, bufs_b, True)

        @pl.when((q > 0) & (ge < nk))
        def _():
            process(ge, bufs_a, bufs_b, False)

        @pl.when(go < nk)
        def _():
            process(go, bufs_b, bufs_a, False)
        return 0

    lax.fori_loop(0, (nk + 1) // 2, step, 0)
    @pl.when(lax.rem(nk, 2) == 1)
    def _():
        wait_scatter(bufs_a)

    @pl.when(lax.rem(nk, 2) == 0)
    def _():
        wait_scatter(bufs_b)

    plsc.subcore_barrier()
    _drain_acc(sid, acc_sh, acc_r.at[cid])


def _run_sc2(src3b, dst3b, as2, ad2, a2dup, h2):
    mesh = plsc.VectorSubcoreMesh(core_axis_name="c", subcore_axis_name="s",
                                  num_cores=NC, num_subcores=NS)
    dbuf = [
        pltpu.VMEM((C2, ACC2W), jnp.float32),
        pltpu.VMEM((C2,), jnp.float32),
        pltpu.SemaphoreType.DMA,
        pltpu.SemaphoreType.DMA,
    ]
    f = pl.kernel(
        _sc_edge2,
        out_type=jax.ShapeDtypeStruct((NC, N, ACC2W), jnp.float32),
        mesh=mesh,
        compiler_params=_SC_PARAMS,
        scratch_types=[
            pltpu.VMEM((NK2MAX, 1, C2), jnp.int32),
            pltpu.VMEM((NK2MAX, 1, C2), jnp.int32),
            pltpu.VMEM((N,), jnp.float32),
            pltpu.VMEM((N,), jnp.float32),
            *dbuf, *dbuf,
            pltpu.VMEM((1, 16), jnp.float32),
            pltpu.VMEM_SHARED((N, ACC2W), jnp.float32),
        ],
    )
    return f(src3b, dst3b, as2, ad2, a2dup, h2)


def _tc_e(acc_ref, b2_ref, out_ref):
    a = acc_ref[0] + acc_ref[1]
    den = a[:, 40:41]
    o = a[:, :40] / (den + 1e-16) + b2_ref[...]
    mx = jnp.max(o, axis=1, keepdims=True)
    o = o - mx
    o = o - jnp.log(jnp.sum(jnp.exp(o), axis=1, keepdims=True))
    out_ref[...] = o


def _run_e(acc2, b2):
    return pl.pallas_call(
        _tc_e,
        grid=(N // RB,),
        in_specs=[
            pl.BlockSpec((NC, RB, ACC2W), lambda i: (0, i, 0)),
            pl.BlockSpec((1, OUT), lambda i: (0, 0)),
        ],
        out_specs=pl.BlockSpec((RB, OUT), lambda i: (i, 0)),
        out_shape=jax.ShapeDtypeStruct((N, OUT), jnp.float32),
    )(acc2, b2)


def kernel(x, edge_index, W1, att_src1, att_dst1, b1, W2, att_src2, att_dst2, b2):
    src3 = edge_index[0].reshape(NCHUNKS, 1, C)
    dst3 = edge_index[1].reshape(NCHUNKS, 1, C)
    src3b = edge_index[0].reshape(E // 128, 1, 128)
    dst3b = edge_index[1].reshape(E // 128, 1, 128)

    eye8 = jnp.eye(HEADS, dtype=jnp.float32)
    S = jnp.einsum("hj,hk->hjk", att_src1, eye8).reshape(IN, HEADS)
    D = jnp.einsum("hj,hk->hjk", att_dst1, eye8).reshape(IN, HEADS)
    SD = jnp.concatenate([S, D], axis=1)

    h1, t1, amax1 = _run_a(x, W1, SD)

    accf, acce = _run_sc1(src3, dst3, t1, h1, amax1)

    W2p = jnp.pad(W2, ((0, 0), (0, ACC2W - OUT)))
    att2p = jnp.zeros((ACC2W, 16), jnp.float32)
    att2p = att2p.at[:OUT, 0].set(att_src2[0])
    att2p = att2p.at[:OUT, 1].set(att_dst2[0])

    h2, t2, amax2 = _run_c(accf, acce, b1.reshape(1, IN), W2p, att2p)
    as2 = t2[:, 0]
    ad2 = t2[:, 1]

    acc2 = _run_sc2(src3b, dst3b, as2, ad2, amax2, h2)

    return _run_e(acc2, b2.reshape(1, OUT))


# speedup vs baseline: 1.0061x; 1.0061x over previous
import jax
import jax.numpy as jnp
from jax import lax
from jax.experimental import pallas as pl
from jax.experimental.pallas import tpu as pltpu
from jax.experimental.pallas import tpu_sc as plsc

N = 10000
E = 320000
IN = 128
HID = 16
HEADS = 8
OUT = 40

NC = 2
NS = 16
NW = NC * NS
C = 80
NCHUNKS = E // C
NK = NCHUNKS // NW
RPT = 624
TAIL0 = RPT * NS
TAILN = N - TAIL0
ACC2W = 48
RB = 2000

_SC_PARAMS = pltpu.CompilerParams(needs_layout_passes=False,
                                  use_tc_tiling_on_sc=False)


def _shuffle16(v, idx16):
    dnums = lax.GatherDimensionNumbers(
        offset_dims=(), collapsed_slice_dims=(0,), start_index_map=(0,))
    return lax.gather(v, idx16[:, None], dnums, (1,),
                      mode=lax.GatherScatterMode.PROMISE_IN_BOUNDS)


def _zero_acc(sid, buf, acc_sh):
    zero16 = jnp.zeros((16,), jnp.float32)
    width = buf.shape[1]

    def zbuf(i, _):
        for k in range(width // 16):
            buf[i, pl.ds(16 * k, 16)] = zero16
        return 0
    lax.fori_loop(0, C, zbuf, 0)

    row0 = sid * RPT
    for off, ln in ((0, 80), (80, 80), (160, 80), (240, 80),
                    (320, 80), (400, 80), (480, 80), (560, 64)):
        pltpu.sync_copy(buf.at[pl.ds(0, ln)], acc_sh.at[pl.ds(row0 + off, ln)])

    @pl.when(sid == NS - 1)
    def _():
        pltpu.sync_copy(buf.at[pl.ds(0, TAILN)], acc_sh.at[pl.ds(TAIL0, TAILN)])


def _drain_acc(sid, acc_sh, out_slab):
    row0 = sid * RPT
    pltpu.sync_copy(acc_sh.at[pl.ds(row0, RPT)], out_slab.at[pl.ds(row0, RPT)])

    @pl.when(sid == NS - 1)
    def _():
        pltpu.sync_copy(acc_sh.at[pl.ds(TAIL0, TAILN)],
                        out_slab.at[pl.ds(TAIL0, TAILN)])


def _tc_a(x_ref, w1_ref, sd_ref, h_ref, t_ref, amax_ref):
    h = jnp.dot(x_ref[...], w1_ref[...], preferred_element_type=jnp.float32)
    h_ref[...] = h
    t = jnp.dot(h, sd_ref[...], preferred_element_type=jnp.float32)
    t_ref[...] = t

    @pl.when(pl.program_id(0) == 0)
    def _():
        amax_ref[...] = jnp.full((1, 16), -1e30, jnp.float32)

    amax_ref[...] = jnp.maximum(amax_ref[...], jnp.max(t, axis=0, keepdims=True))


def _run_a(x, W1, SD):
    return pl.pallas_call(
        _tc_a,
        grid=(N // RB,),
        in_specs=[
            pl.BlockSpec((RB, IN), lambda i: (i, 0)),
            pl.BlockSpec((IN, IN), lambda i: (0, 0)),
            pl.BlockSpec((IN, 16), lambda i: (0, 0)),
        ],
        out_specs=[
            pl.BlockSpec((RB, IN), lambda i: (i, 0)),
            pl.BlockSpec((RB, 16), lambda i: (i, 0)),
            pl.BlockSpec((1, 16), lambda i: (0, 0)),
        ],
        out_shape=[
            jax.ShapeDtypeStruct((N, IN), jnp.float32),
            jax.ShapeDtypeStruct((N, 16), jnp.float32),
            jax.ShapeDtypeStruct((1, 16), jnp.float32),
        ],
    )(x, W1, SD)


def _sc_edge1(src_r, dst_r, t1_r, h1_r, adup_r, accf_r, acce_r,
              ids_s3, ids_d3, isem,
              gs_a, gd_a, hrow_a, exh_a, exb_a, sem_a, ssem_a,
              gs_b, gd_b, hrow_b, exh_b, exb_b, sem_b, ssem_b,
              adup_v, accf_sh, acce_sh):
    cid = lax.axis_index("c")
    sid = lax.axis_index("s")
    wid = sid * NC + cid
    g0 = wid * NK

    _zero_acc(sid, hrow_a, accf_sh)
    _zero_acc(sid, exh_a, acce_sh)

    iota = lax.iota(jnp.int32, 16)
    lo8 = iota < 8
    col_s = iota & 7
    col_d = col_s + 8
    hi8 = (iota >= 8).astype(jnp.int32)
    pltpu.sync_copy(adup_r, adup_v)
    adv = _shuffle16(adup_v[0], col_s)
    plsc.subcore_barrier()

    bufs_a = (gs_a, gd_a, hrow_a, exh_a, exb_a, sem_a, ssem_a)
    bufs_b = (gs_b, gd_b, hrow_b, exh_b, exb_b, sem_b, ssem_b)

    def fire_ids(g):
        slot = lax.rem(g, 3)
        pltpu.async_copy(src_r.at[pl.ds(g0 + g, 1)],
                         ids_s3.at[pl.ds(slot, 1)], isem)
        pltpu.async_copy(dst_r.at[pl.ds(g0 + g, 1)],
                         ids_d3.at[pl.ds(slot, 1)], isem)

    def wait_ids():
        pltpu.make_async_copy(src_r.at[pl.ds(0, 1)],
                              ids_s3.at[pl.ds(0, 1)], isem).wait()
        pltpu.make_async_copy(dst_r.at[pl.ds(0, 1)],
                              ids_d3.at[pl.ds(0, 1)], isem).wait()

    def fire_gathers(g, bufs):
        gs, gd, hrow, _, _, sem, _ = bufs
        slot = lax.rem(g, 3)
        pltpu.async_copy(t1_r.at[ids_s3.at[slot, 0]], gs, sem)
        pltpu.async_copy(t1_r.at[ids_d3.at[slot, 0]], gd, sem)
        pltpu.async_copy(h1_r.at[ids_s3.at[slot, 0]], hrow, sem)

    def wait_gathers(bufs):
        gs, gd, hrow, _, _, sem, _ = bufs
        pltpu.make_async_copy(t1_r.at[ids_s3.at[0, 0]], gs, sem).wait()
        pltpu.make_async_copy(t1_r.at[ids_d3.at[0, 0]], gd, sem).wait()
        pltpu.make_async_copy(h1_r.at[ids_s3.at[0, 0]], hrow, sem).wait()

    def fire_scatter(g, bufs):
        _, _, hrow, exh, _, _, ssem = bufs
        slot = lax.rem(g, 3)
        pltpu.async_copy(hrow, accf_sh.at[ids_d3.at[slot, 0]], ssem, add=True)
        pltpu.async_copy(exh, acce_sh.at[ids_d3.at[slot, 0]], ssem, add=True)

    def wait_scatter(bufs):
        _, _, hrow, exh, _, _, ssem = bufs
        pltpu.make_async_copy(hrow, accf_sh.at[ids_d3.at[0, 0]], ssem).wait()
        pltpu.make_async_copy(exh, acce_sh.at[ids_d3.at[0, 0]], ssem).wait()

    def compute(bufs):
        gs, gd, hrow, exh, exb, _, _ = bufs

        def ex_pair(jp):
            row_idx = 2 * jp + hi8
            s_v = plsc.load_gather(gs, [row_idx, col_s])
            d_v = plsc.load_gather(gd, [row_idx, col_d])
            e = s_v + d_v
            el = jnp.maximum(e, 0.2 * e)
            ad = adv + d_v
            m = jnp.maximum(ad, 0.2 * ad)
            exv = jnp.exp(el - m)
            exb[pl.ds(16 * jp, 16)] = exv
            exh[2 * jp, pl.ds(0, 16)] = jnp.where(lo8, exv, 0.0)
            shuf = _shuffle16(exv, col_d)
            exh[2 * jp + 1, pl.ds(0, 16)] = jnp.where(lo8, shuf, 0.0)

        @plsc.parallel_loop(0, C // 2, unroll=2)
        def _(jp):
            ex_pair(jp)

        def m_pair(jp):
            ex16 = exb[pl.ds(16 * jp, 16)]
            ia = 2 * jp
            ib = 2 * jp + 1
            for k2 in range(8):
                hrow[ia, pl.ds(16 * k2, 16)] = (
                    hrow[ia, pl.ds(16 * k2, 16)] * ex16[k2])
                hrow[ib, pl.ds(16 * k2, 16)] = (
                    hrow[ib, pl.ds(16 * k2, 16)] * ex16[8 + k2])

        @plsc.parallel_loop(0, C // 2, unroll=2)
        def _(jp):
            m_pair(jp)

    pltpu.sync_copy(src_r.at[pl.ds(g0, 1)], ids_s3.at[pl.ds(0, 1)])
    pltpu.sync_copy(dst_r.at[pl.ds(g0, 1)], ids_d3.at[pl.ds(0, 1)])
    fire_gathers(0, bufs_a)
    fire_ids(1)

    def process(g, cur, oth, first):
        wait_gathers(cur)
        if not first:
            wait_scatter(oth)

        @pl.when(g + 1 < NK)
        def _():
            wait_ids()
            fire_gathers(g + 1, oth)

        @pl.when(g + 2 < NK)
        def _():
            fire_ids(g + 2)
        compute(cur)
        fire_scatter(g, cur)

    def step(q, _):
        ge = 2 * q
        go = 2 * q + 1

        @pl.when(q == 0)
        def _():
            process(0, bufs_a, bufs_b, True)

        @pl.when(q > 0)
        def _():
            process(ge, bufs_a, bufs_b, False)

        @pl.when(go < NK)
        def _():
            process(go, bufs_b, bufs_a, False)
        return 0

    lax.fori_loop(0, (NK + 1) // 2, step, 0)
    wait_scatter(bufs_a)

    plsc.subcore_barrier()
    _drain_acc(sid, accf_sh, accf_r.at[cid])
    _drain_acc(sid, acce_sh, acce_r.at[cid])


def _run_sc1(src3, dst3, t1, h1, adup):
    mesh = plsc.VectorSubcoreMesh(core_axis_name="c", subcore_axis_name="s",
                                  num_cores=NC, num_subcores=NS)
    dbuf = [
        pltpu.VMEM((C, 16), jnp.float32),
        pltpu.VMEM((C, 16), jnp.float32),
        pltpu.VMEM((C, IN), jnp.float32),
        pltpu.VMEM((C, 16), jnp.float32),
        pltpu.VMEM((C * 8,), jnp.float32),
        pltpu.SemaphoreType.DMA,
        pltpu.SemaphoreType.DMA,
    ]
    f = pl.kernel(
        _sc_edge1,
        out_type=[
            jax.ShapeDtypeStruct((NC, N, IN), jnp.float32),
            jax.ShapeDtypeStruct((NC, N, 16), jnp.float32),
        ],
        mesh=mesh,
        compiler_params=_SC_PARAMS,
        scratch_types=[
            pltpu.VMEM((3, 1, C), jnp.int32),
            pltpu.VMEM((3, 1, C), jnp.int32),
            pltpu.SemaphoreType.DMA,
            *dbuf, *dbuf,
            pltpu.VMEM((1, 16), jnp.float32),
            pltpu.VMEM_SHARED((N, IN), jnp.float32),
            pltpu.VMEM_SHARED((N, 16), jnp.float32),
        ],
    )
    return f(src3, dst3, t1, h1, adup)


def _tc_c(accf_ref, acce_ref, b1_ref, w2p_ref, a2p_ref, h2_ref, t2_ref, amax_ref):
    a = accf_ref[0] + accf_ref[1]
    den8 = (acce_ref[0] + acce_ref[1])[:, :8]
    r8 = lax.broadcasted_iota(jnp.int32, (8, 128), 0)
    c8 = lax.broadcasted_iota(jnp.int32, (8, 128), 1)
    expand = (r8 == c8 // 16).astype(jnp.float32)
    denx = jnp.dot(den8, expand, preferred_element_type=jnp.float32)
    o = a / (denx + 1e-16) + b1_ref[...]
    o = jnp.where(o > 0, o, jnp.exp(jnp.minimum(o, 0.0)) - 1.0)
    h2 = jnp.dot(o, w2p_ref[...], preferred_element_type=jnp.float32)
    c48 = (lax.broadcasted_iota(jnp.int32, (1, ACC2W), 1) == 40).astype(jnp.float32)
    h2 = h2 + c48
    h2_ref[...] = h2
    t2 = jnp.dot(h2, a2p_ref[...], preferred_element_type=jnp.float32)
    t2_ref[...] = t2

    @pl.when(pl.program_id(0) == 0)
    def _():
        amax_ref[...] = jnp.full((1, 16), -1e30, jnp.float32)

    amax_ref[...] = jnp.maximum(amax_ref[...], jnp.max(t2, axis=0, keepdims=True))


def _run_c(accf, acce, b1, W2p, att2p):
    return pl.pallas_call(
        _tc_c,
        grid=(N // RB,),
        in_specs=[
            pl.BlockSpec((NC, RB, IN), lambda i: (0, i, 0)),
            pl.BlockSpec((NC, RB, 16), lambda i: (0, i, 0)),
            pl.BlockSpec((1, 128), lambda i: (0, 0)),
            pl.BlockSpec((128, ACC2W), lambda i: (0, 0)),
            pl.BlockSpec((ACC2W, 16), lambda i: (0, 0)),
        ],
        out_specs=[
            pl.BlockSpec((RB, ACC2W), lambda i: (i, 0)),
            pl.BlockSpec((RB, 16), lambda i: (i, 0)),
            pl.BlockSpec((1, 16), lambda i: (0, 0)),
        ],
        out_shape=[
            jax.ShapeDtypeStruct((N, ACC2W), jnp.float32),
            jax.ShapeDtypeStruct((N, 16), jnp.float32),
            jax.ShapeDtypeStruct((1, 16), jnp.float32),
        ],
    )(accf, acce, b1, W2p, att2p)


C2 = 128
NCHUNKS2 = E // C2
NK2MIN = NCHUNKS2 // NW
NEXTRA2 = NCHUNKS2 - NK2MIN * NW
NK2MAX = NK2MIN + 1


def _sc_edge2(src_r, dst_r, as2_r, ad2_r, a2dup_r, h2_r, acc_r,
              ids_s3, ids_d3, as2_t, ad2_t,
              h2b_a, exb2_a, sem_a, ssem_a,
              h2b_b, exb2_b, sem_b, ssem_b,
              a2v, acc_sh):
    cid = lax.axis_index("c")
    sid = lax.axis_index("s")
    wid = sid * NC + cid
    g0 = NK2MIN * wid + jnp.minimum(wid, NEXTRA2)
    nk = NK2MIN + jnp.where(wid < NEXTRA2, 1, 0)

    _zero_acc(sid, h2b_a, acc_sh)
    pltpu.sync_copy(src_r.at[pl.ds(g0, NK2MIN)], ids_s3.at[pl.ds(0, NK2MIN)])
    pltpu.sync_copy(dst_r.at[pl.ds(g0, NK2MIN)], ids_d3.at[pl.ds(0, NK2MIN)])

    @pl.when(nk > NK2MIN)
    def _():
        pltpu.sync_copy(src_r.at[pl.ds(g0 + NK2MIN, 1)],
                        ids_s3.at[pl.ds(NK2MIN, 1)])
        pltpu.sync_copy(dst_r.at[pl.ds(g0 + NK2MIN, 1)],
                        ids_d3.at[pl.ds(NK2MIN, 1)])
    pltpu.sync_copy(as2_r, as2_t)
    pltpu.sync_copy(ad2_r, ad2_t)
    pltpu.sync_copy(a2dup_r, a2v)
    plsc.subcore_barrier()

    a2 = _shuffle16(a2v[0], jnp.zeros((16,), jnp.int32))
    bufs_a = (h2b_a, exb2_a, sem_a, ssem_a)
    bufs_b = (h2b_b, exb2_b, sem_b, ssem_b)

    def fire_gathers(g, bufs):
        h2b, _, sem, _ = bufs
        pltpu.async_copy(h2_r.at[ids_s3.at[g, 0]], h2b, sem)

    def wait_gathers(bufs):
        h2b, _, sem, _ = bufs
        pltpu.make_async_copy(h2_r.at[ids_s3.at[0, 0]], h2b, sem).wait()

    def fire_scatter(g, bufs):
        h2b, _, _, ssem = bufs
        pltpu.async_copy(h2b, acc_sh.at[ids_d3.at[g, 0]], ssem, add=True)

    def wait_scatter(bufs):
        h2b, _, _, ssem = bufs
        pltpu.make_async_copy(h2b, acc_sh.at[ids_d3.at[0, 0]], ssem).wait()

    def compute_scale(g, bufs):
        h2b, exb2, _, _ = bufs

        def exloop(j):
            sv = ids_s3[g, 0, pl.ds(16 * j, 16)]
            dv = ids_d3[g, 0, pl.ds(16 * j, 16)]
            s = plsc.load_gather(as2_t, [sv])
            d = plsc.load_gather(ad2_t, [dv])
            e = s + d
            el = jnp.maximum(e, 0.2 * e)
            ad = a2 + d
            m = jnp.maximum(ad, 0.2 * ad)
            exb2[pl.ds(16 * j, 16)] = jnp.exp(el - m)
        plsc.parallel_loop(0, C2 // 16, unroll=2)(exloop)

        @plsc.parallel_loop(0, C2 // 16, unroll=2)
        def _(j):
            ex16 = exb2[pl.ds(16 * j, 16)]
            for k2 in range(16):
                i = 16 * j + k2
                sc = ex16[k2]
                for b in range(ACC2W // 16):
                    h2b[i, pl.ds(16 * b, 16)] = h2b[i, pl.ds(16 * b, 16)] * sc

    fire_gathers(0, bufs_a)

    def process(g, cur, oth, first):
        wait_gathers(cur)
        if not first:
            wait_scatter(oth)

        @pl.when(g + 1 < nk)
        def _():
            fire_gathers(g + 1, oth)
        compute_scale(g, cur)
        fire_scatter(g, cur)

    def step(q, _):
        ge = 2 * q
        go = 2 * q + 1

        @pl.when(q == 0)
        def _():
            process(0, bufs_a, bufs_b, True)

        @pl.when((q > 0) & (ge < nk))
        def _():
            process(ge, bufs_a, bufs_b, False)

        @pl.when(go < nk)
        def _():
            process(go, bufs_b, bufs_a, False)
        return 0

    lax.fori_loop(0, (nk + 1) // 2, step, 0)
    @pl.when(lax.rem(nk, 2) == 1)
    def _():
        wait_scatter(bufs_a)

    @pl.when(lax.rem(nk, 2) == 0)
    def _():
        wait_scatter(bufs_b)

    plsc.subcore_barrier()
    _drain_acc(sid, acc_sh, acc_r.at[cid])


def _run_sc2(src3b, dst3b, as2, ad2, a2dup, h2):
    mesh = plsc.VectorSubcoreMesh(core_axis_name="c", subcore_axis_name="s",
                                  num_cores=NC, num_subcores=NS)
    dbuf = [
        pltpu.VMEM((C2, ACC2W), jnp.float32),
        pltpu.VMEM((C2,), jnp.float32),
        pltpu.SemaphoreType.DMA,
        pltpu.SemaphoreType.DMA,
    ]
    f = pl.kernel(
        _sc_edge2,
        out_type=jax.ShapeDtypeStruct((NC, N, ACC2W), jnp.float32),
        mesh=mesh,
        compiler_params=_SC_PARAMS,
        scratch_types=[
            pltpu.VMEM((NK2MAX, 1, C2), jnp.int32),
            pltpu.VMEM((NK2MAX, 1, C2), jnp.int32),
            pltpu.VMEM((N,), jnp.float32),
            pltpu.VMEM((N,), jnp.float32),
            *dbuf, *dbuf,
            pltpu.VMEM((1, 16), jnp.float32),
            pltpu.VMEM_SHARED((N, ACC2W), jnp.float32),
        ],
    )
    return f(src3b, dst3b, as2, ad2, a2dup, h2)


def _tc_e(acc_ref, b2_ref, out_ref):
    a = acc_ref[0] + acc_ref[1]
    den = a[:, 40:41]
    o = a[:, :40] / (den + 1e-16) + b2_ref[...]
    mx = jnp.max(o, axis=1, keepdims=True)
    o = o - mx
    o = o - jnp.log(jnp.sum(jnp.exp(o), axis=1, keepdims=True))
    out_ref[...] = o


def _run_e(acc2, b2):
    return pl.pallas_call(
        _tc_e,
        grid=(N // RB,),
        in_specs=[
            pl.BlockSpec((NC, RB, ACC2W), lambda i: (0, i, 0)),
            pl.BlockSpec((1, OUT), lambda i: (0, 0)),
        ],
        out_specs=pl.BlockSpec((RB, OUT), lambda i: (i, 0)),
        out_shape=jax.ShapeDtypeStruct((N, OUT), jnp.float32),
    )(acc2, b2)


def kernel(x, edge_index, W1, att_src1, att_dst1, b1, W2, att_src2, att_dst2, b2):
    src3 = edge_index[0].reshape(NCHUNKS, 1, C)
    dst3 = edge_index[1].reshape(NCHUNKS, 1, C)
    src3b = edge_index[0].reshape(E // 128, 1, 128)
    dst3b = edge_index[1].reshape(E // 128, 1, 128)

    eye8 = jnp.eye(HEADS, dtype=jnp.float32)
    S = jnp.einsum("hj,hk->hjk", att_src1, eye8).reshape(IN, HEADS)
    D = jnp.einsum("hj,hk->hjk", att_dst1, eye8).reshape(IN, HEADS)
    SD = jnp.concatenate([S, D], axis=1)

    h1, t1, amax1 = _run_a(x, W1, SD)

    accf, acce = _run_sc1(src3, dst3, t1, h1, amax1)

    W2p = jnp.pad(W2, ((0, 0), (0, ACC2W - OUT)))
    att2p = jnp.zeros((ACC2W, 16), jnp.float32)
    att2p = att2p.at[:OUT, 0].set(att_src2[0])
    att2p = att2p.at[:OUT, 1].set(att_dst2[0])

    h2, t2, amax2 = _run_c(accf, acce, b1.reshape(1, IN), W2p, att2p)
    as2 = t2[:, 0]
    ad2 = t2[:, 1]

    acc2 = _run_sc2(src3b, dst3b, as2, ad2, amax2, h2)

    return _run_e(acc2, b2.reshape(1, OUT))
